# Initial kernel scaffold; baseline (speedup 1.0000x reference)
import jax
import jax.numpy as jnp
from jax.experimental import pallas as pl


def kernel(X, A, W_down0, b_down0, W_down1, b_down1, W_down2, b_down2, W_down3, b_down3, p_pool0, p_pool1, p_pool2, W_up0, b_up0, W_up1, b_up1, W_up2, b_up2):
    raise NotImplementedError("write your pallas kernel here")


# speedup vs baseline: 4778.9163x; 4778.9163x over previous
import jax
import jax.numpy as jnp
from jax import lax
from jax.experimental import pallas as pl

N = 1024
F32 = jnp.float32


def _mm(a, b, ca, cb):
    return lax.dot_general(a, b, (((ca,), (cb,)), ((), ())),
                           preferred_element_type=F32)


def _gunet_body(X_ref, A_ref,
                Wd0_ref, bd0_ref, Wd1_ref, bd1_ref, Wd2_ref, bd2_ref,
                Wd3_ref, bd3_ref, p0_ref, p1_ref, p2_ref,
                Wu0_ref, bu0_ref, Wu1_ref, bu1_ref, Wu2_ref, bu2_ref,
                out_ref):
    X = X_ref[...]
    A = A_ref[...]

    ri = lax.broadcasted_iota(jnp.int32, (N, N), 0)
    ci = lax.broadcasted_iota(jnp.int32, (N, N), 1)
    eye_b = ri == ci
    eye_f = eye_b.astype(F32)
    ones_col = jnp.ones((N, 1), F32)

    def row_of(col):
        return jnp.sum(eye_f * col, axis=0, keepdims=True)

    def gcn(x, Aadj, W, b):
        deg = _mm(Aadj, ones_col, 0, 0) + 2.0
        dis = 1.0 / jnp.sqrt(deg)
        h = _mm(x, W, 1, 0)
        core = _mm(Aadj, dis * h, 0, 0)
        return dis * core + (2.0 * dis * dis) * h + b

    def pool(x, Acur, mask, p, k):
        Ap = jnp.where(eye_b, mask, Acur)
        A2 = jnp.where(eye_b, 0.0, _mm(Ap, Ap, 1, 0))
        nrm = jnp.sqrt(_mm(p, p, 1, 1))
        s = jnp.tanh(_mm(x, p, 1, 1) / nrm)
        sm = jnp.where(mask > 0, s, -2.0)
        sm_row = row_of(sm)
        beats = (sm_row > sm) | ((sm_row == sm) & (ci < ri))
        rank = _mm(beats.astype(F32), mask, 1, 0)
        nmask = ((rank < k) & (mask > 0)).astype(F32)
        xp = x * s * nmask
        Anew = A2 * nmask * row_of(nmask)
        return xp, Anew, nmask

    ew0 = jnp.where(A != 0, 1.0, 0.0).astype(F32)
    m0 = ones_col
    x0 = jax.nn.relu(gcn(X, ew0, Wd0_ref[...], bd0_ref[...]))

    xp, A1, m1 = pool(x0, ew0, m0, p0_ref[...], 512)
    x1 = jax.nn.relu(gcn(xp, A1, Wd1_ref[...], bd1_ref[...]))

    xp, A2_, m2 = pool(x1, A1, m1, p1_ref[...], 256)
    x2 = jax.nn.relu(gcn(xp, A2_, Wd2_ref[...], bd2_ref[...]))

    xp, A3, m3 = pool(x2, A2_, m2, p2_ref[...], 128)
    x3 = jax.nn.relu(gcn(xp, A3, Wd3_ref[...], bd3_ref[...]))

    x = x2 + x3 * m3
    x = jax.nn.relu(gcn(x, A2_, Wu0_ref[...], bu0_ref[...]))
    x = x1 + x * m2
    x = jax.nn.relu(gcn(x, A1, Wu1_ref[...], bu1_ref[...]))
    x = x0 + x * m1
    out_ref[...] = gcn(x, ew0, Wu2_ref[...], bu2_ref[...])


@jax.jit
def kernel(X, A, W_down0, b_down0, W_down1, b_down1, W_down2, b_down2,
           W_down3, b_down3, p_pool0, p_pool1, p_pool2,
           W_up0, b_up0, W_up1, b_up1, W_up2, b_up2):
    args = (
        X, A,
        W_down0, b_down0.reshape(1, -1),
        W_down1, b_down1.reshape(1, -1),
        W_down2, b_down2.reshape(1, -1),
        W_down3, b_down3.reshape(1, -1),
        p_pool0.reshape(1, -1), p_pool1.reshape(1, -1), p_pool2.reshape(1, -1),
        W_up0, b_up0.reshape(1, -1),
        W_up1, b_up1.reshape(1, -1),
        W_up2, b_up2.reshape(1, -1),
    )
    return pl.pallas_call(
        _gunet_body,
        out_shape=jax.ShapeDtypeStruct((N, W_up2.shape[1]), F32),
    )(*args)


# speedup vs baseline: 4809.8594x; 1.0065x over previous
import jax
import jax.numpy as jnp
from jax import lax
from jax.experimental import pallas as pl

N = 1024
F32 = jnp.float32


def _mm(a, b, ca, cb):
    return lax.dot_general(a, b, (((ca,), (cb,)), ((), ())),
                           preferred_element_type=F32)


def _gunet_body(X_ref, A_ref,
                Wd0_ref, bd0_ref, Wd1_ref, bd1_ref, Wd2_ref, bd2_ref,
                Wd3_ref, bd3_ref, p0_ref, p1_ref, p2_ref,
                Wu0_ref, bu0_ref, Wu1_ref, bu1_ref, Wu2_ref, bu2_ref,
                out_ref):
    X = X_ref[...]
    A = A_ref[...]

    ri = lax.broadcasted_iota(jnp.int32, (N, N), 0)
    ci = lax.broadcasted_iota(jnp.int32, (N, N), 1)
    eye_b = ri == ci
    eye_f = eye_b.astype(F32)
    ones_col = jnp.ones((N, 1), F32)

    def row_of(col):
        return jnp.sum(eye_f * col, axis=0, keepdims=True)

    def gcn(x, Aadj, W, b):
        deg = _mm(Aadj, ones_col, 0, 0) + 2.0
        dis = 1.0 / jnp.sqrt(deg)
        h = _mm(x, W, 1, 0)
        core = _mm(Aadj, dis * h, 0, 0)
        return dis * core + (2.0 * dis * dis) * h + b

    def pool(x, Acur, mask, p, k, binary=False):
        Ap = jnp.where(eye_b, mask, Acur)
        if binary:
            Ap = Ap.astype(jnp.bfloat16)
        A2 = jnp.where(eye_b, 0.0, _mm(Ap, Ap, 1, 0))
        nrm = jnp.sqrt(_mm(p, p, 1, 1))
        s = jnp.tanh(_mm(x, p, 1, 1) / nrm)
        sm = jnp.where(mask > 0, s, -2.0)
        sm_row = row_of(sm)
        beats = (sm_row > sm) | ((sm_row == sm) & (ci < ri))
        rank = _mm(beats.astype(jnp.bfloat16),
                   mask.astype(jnp.bfloat16), 1, 0)
        nmask = ((rank < k) & (mask > 0)).astype(F32)
        xp = x * s * nmask
        Anew = A2 * nmask * row_of(nmask)
        return xp, Anew, nmask

    ew0 = jnp.where(A != 0, 1.0, 0.0).astype(F32)
    m0 = ones_col
    x0 = jax.nn.relu(gcn(X, ew0, Wd0_ref[...], bd0_ref[...]))

    xp, A1, m1 = pool(x0, ew0, m0, p0_ref[...], 512, binary=True)
    x1 = jax.nn.relu(gcn(xp, A1, Wd1_ref[...], bd1_ref[...]))

    xp, A2_, m2 = pool(x1, A1, m1, p1_ref[...], 256)
    x2 = jax.nn.relu(gcn(xp, A2_, Wd2_ref[...], bd2_ref[...]))

    xp, A3, m3 = pool(x2, A2_, m2, p2_ref[...], 128)
    x3 = jax.nn.relu(gcn(xp, A3, Wd3_ref[...], bd3_ref[...]))

    x = x2 + x3 * m3
    x = jax.nn.relu(gcn(x, A2_, Wu0_ref[...], bu0_ref[...]))
    x = x1 + x * m2
    x = jax.nn.relu(gcn(x, A1, Wu1_ref[...], bu1_ref[...]))
    x = x0 + x * m1
    out_ref[...] = gcn(x, ew0, Wu2_ref[...], bu2_ref[...])


@jax.jit
def kernel(X, A, W_down0, b_down0, W_down1, b_down1, W_down2, b_down2,
           W_down3, b_down3, p_pool0, p_pool1, p_pool2,
           W_up0, b_up0, W_up1, b_up1, W_up2, b_up2):
    args = (
        X, A,
        W_down0, b_down0.reshape(1, -1),
        W_down1, b_down1.reshape(1, -1),
        W_down2, b_down2.reshape(1, -1),
        W_down3, b_down3.reshape(1, -1),
        p_pool0.reshape(1, -1), p_pool1.reshape(1, -1), p_pool2.reshape(1, -1),
        W_up0, b_up0.reshape(1, -1),
        W_up1, b_up1.reshape(1, -1),
        W_up2, b_up2.reshape(1, -1),
    )
    return pl.pallas_call(
        _gunet_body,
        out_shape=jax.ShapeDtypeStruct((N, W_up2.shape[1]), F32),
    )(*args)


# speedup vs baseline: 4908.5498x; 1.0205x over previous
import jax
import jax.numpy as jnp
from jax import lax
from jax.experimental import pallas as pl

N = 1024
F32 = jnp.float32
BF16 = jnp.bfloat16


def _mm(a, b, ca, cb):
    return lax.dot_general(a, b, (((ca,), (cb,)), ((), ())),
                           preferred_element_type=F32)


def _gunet_body(X_ref, A_ref,
                Wd0_ref, bd0_ref, Wd1_ref, bd1_ref, Wd2_ref, bd2_ref,
                Wd3_ref, bd3_ref, p0_ref, p1_ref, p2_ref,
                Wu0_ref, bu0_ref, Wu1_ref, bu1_ref, Wu2_ref, bu2_ref,
                out_ref):
    X = X_ref[...]
    A = A_ref[...]

    ri = lax.broadcasted_iota(jnp.int32, (N, N), 0)
    ci = lax.broadcasted_iota(jnp.int32, (N, N), 1)
    eye_b = ri == ci
    lt_b = ci < ri
    ones_col = jnp.ones((N, 1), F32)

    def row_of(col):
        return jnp.transpose(col)

    def dis_of(Aadj):
        deg = lax.dot_general(Aadj, ones_col, (((0,), (0,)), ((), ())), preferred_element_type=F32) + 2.0
        return 1.0 / jnp.sqrt(deg)

    def gcn(x, Aadj, dis, W, b):
        h = lax.dot_general(x, W, (((1,), (0,)), ((), ())), preferred_element_type=F32)
        core = lax.dot_general(Aadj, dis * h, (((0,), (0,)), ((), ())), preferred_element_type=F32)
        return dis * core + (2.0 * dis * dis) * h + b

    def pool(x, Acur, mask, p, k, binary=False):
        Ap = jnp.where(eye_b, mask, Acur)
        if binary:
            Ap = Ap.astype(BF16)
        A2 = lax.dot_general(Ap, Ap, (((1,), (0,)), ((), ())), preferred_element_type=F32)
        nrm = jnp.sqrt(_mm(p, p, 1, 1))
        s = jnp.tanh(lax.dot_general(x, p, (((1,), (1,)), ((), ())), preferred_element_type=F32) / nrm)
        sm = jnp.where(mask > 0, s, -2.0)
        sm_row = row_of(sm)
        beats = (sm_row > sm) | ((sm_row == sm) & lt_b)
        rank = lax.dot_general(beats.astype(BF16), mask.astype(BF16),
                               (((1,), (0,)), ((), ())), preferred_element_type=F32)
        nmask = ((rank < k) & (mask > 0)).astype(F32)
        xp = x * s * nmask
        Anew = jnp.where(eye_b, 0.0, A2 * nmask * row_of(nmask))
        return xp, Anew, nmask

    ew0 = jnp.where(A != 0, 1.0, 0.0).astype(F32)
    m0 = ones_col
    dis0 = dis_of(ew0)
    x0 = jax.nn.relu(gcn(X, ew0, dis0, Wd0_ref[...], bd0_ref[...]))

    xp, A1, m1 = pool(x0, ew0, m0, p0_ref[...], 512, binary=True)
    dis1 = dis_of(A1)
    x1 = jax.nn.relu(gcn(xp, A1, dis1, Wd1_ref[...], bd1_ref[...]))

    xp, A2_, m2 = pool(x1, A1, m1, p1_ref[...], 256)
    dis2 = dis_of(A2_)
    x2 = jax.nn.relu(gcn(xp, A2_, dis2, Wd2_ref[...], bd2_ref[...]))

    xp, A3, m3 = pool(x2, A2_, m2, p2_ref[...], 128)
    dis3 = dis_of(A3)
    x3 = jax.nn.relu(gcn(xp, A3, dis3, Wd3_ref[...], bd3_ref[...]))

    x = x2 + x3 * m3
    x = jax.nn.relu(gcn(x, A2_, dis2, Wu0_ref[...], bu0_ref[...]))
    x = x1 + x * m2
    x = jax.nn.relu(gcn(x, A1, dis1, Wu1_ref[...], bu1_ref[...]))
    x = x0 + x * m1
    out_ref[...] = gcn(x, ew0, dis0, Wu2_ref[...], bu2_ref[...])


@jax.jit
def kernel(X, A, W_down0, b_down0, W_down1, b_down1, W_down2, b_down2,
           W_down3, b_down3, p_pool0, p_pool1, p_pool2,
           W_up0, b_up0, W_up1, b_up1, W_up2, b_up2):
    args = (
        X, A,
        W_down0, b_down0.reshape(1, -1),
        W_down1, b_down1.reshape(1, -1),
        W_down2, b_down2.reshape(1, -1),
        W_down3, b_down3.reshape(1, -1),
        p_pool0.reshape(1, -1), p_pool1.reshape(1, -1), p_pool2.reshape(1, -1),
        W_up0, b_up0.reshape(1, -1),
        W_up1, b_up1.reshape(1, -1),
        W_up2, b_up2.reshape(1, -1),
    )
    return pl.pallas_call(
        _gunet_body,
        out_shape=jax.ShapeDtypeStruct((N, W_up2.shape[1]), F32),
    )(*args)


# speedup vs baseline: 6281.0262x; 1.2796x over previous
import jax
import jax.numpy as jnp
from jax import lax
from jax.experimental import pallas as pl

N = 1024
F32 = jnp.float32
BF16 = jnp.bfloat16


def _mm(a, b, ca, cb):
    return lax.dot_general(a, b, (((ca,), (cb,)), ((), ())),
                           preferred_element_type=F32)


def _gunet_body(X_ref, A_ref,
                Wd0_ref, bd0_ref, Wd1_ref, bd1_ref, Wd2_ref, bd2_ref,
                Wd3_ref, bd3_ref, p0_ref, p1_ref, p2_ref,
                Wu0_ref, bu0_ref, Wu1_ref, bu1_ref, Wu2_ref, bu2_ref,
                out_ref):
    X = X_ref[...]
    A = A_ref[...]

    def dis_of(Aadj):
        n = Aadj.shape[0]
        ones_col = jnp.ones((n, 1), F32)
        deg = lax.dot_general(Aadj, ones_col, (((0,), (0,)), ((), ())), preferred_element_type=F32) + 2.0
        return 1.0 / jnp.sqrt(deg)

    def gcn(x, Aadj, dis, W, b):
        h = lax.dot_general(x, W, (((1,), (0,)), ((), ())), preferred_element_type=F32)
        core = lax.dot_general(Aadj, dis * h, (((0,), (0,)), ((), ())), preferred_element_type=F32)
        return dis * core + (2.0 * dis * dis) * h + b

    def pool(x, Acur, p, binary=False):
        n = x.shape[0]
        k = n // 2
        ri = lax.broadcasted_iota(jnp.int32, (n, n), 0)
        ci = lax.broadcasted_iota(jnp.int32, (n, n), 1)
        eye_b = ri == ci
        Ap = jnp.where(eye_b, 1.0, Acur)
        if binary:
            Ap = Ap.astype(BF16)
        A2 = lax.dot_general(Ap, Ap, (((1,), (0,)), ((), ())), preferred_element_type=F32)
        nrm = jnp.sqrt(_mm(p, p, 1, 1))
        s = jnp.tanh(lax.dot_general(x, p, (((1,), (1,)), ((), ())), preferred_element_type=F32) / nrm)
        s_row = jnp.transpose(s)
        beats = ((s_row > s) | ((s_row == s) & (ci < ri))).astype(BF16)
        rank = lax.dot_general(beats, jnp.ones((n, 1), BF16),
                               (((1,), (0,)), ((), ())), preferred_element_type=F32)
        sel = rank < k
        ltri = (ci < ri).astype(BF16)
        pos = lax.dot_general(ltri, sel.astype(BF16),
                              (((1,), (0,)), ((), ())), preferred_element_type=F32)
        rif = ri.astype(F32)
        P = ((rif == jnp.transpose(pos)) & jnp.transpose(sel)).astype(F32)[:k, :]
        xc = lax.dot_general(P, x * s, (((1,), (0,)), ((), ())), preferred_element_type=F32)
        PA2 = lax.dot_general(P, A2, (((1,), (0,)), ((), ())), preferred_element_type=F32)
        Ac = lax.dot_general(PA2, P, (((1,), (1,)), ((), ())), preferred_element_type=F32)
        eye_k = (lax.broadcasted_iota(jnp.int32, (k, k), 0)
                 == lax.broadcasted_iota(jnp.int32, (k, k), 1))
        Ac = jnp.where(eye_k, 0.0, Ac)
        return xc, Ac, P

    def expand(P, xc):
        return lax.dot_general(P, xc, (((0,), (0,)), ((), ())), preferred_element_type=F32)

    ew0 = jnp.where(A != 0, 1.0, 0.0).astype(F32)
    dis0 = dis_of(ew0)
    x0 = jax.nn.relu(gcn(X, ew0, dis0, Wd0_ref[...], bd0_ref[...]))

    xp, A1, P1 = pool(x0, ew0, p0_ref[...], binary=True)
    dis1 = dis_of(A1)
    x1 = jax.nn.relu(gcn(xp, A1, dis1, Wd1_ref[...], bd1_ref[...]))

    xp, A2_, P2 = pool(x1, A1, p1_ref[...])
    dis2 = dis_of(A2_)
    x2 = jax.nn.relu(gcn(xp, A2_, dis2, Wd2_ref[...], bd2_ref[...]))

    xp, A3, P3 = pool(x2, A2_, p2_ref[...])
    dis3 = dis_of(A3)
    x3 = jax.nn.relu(gcn(xp, A3, dis3, Wd3_ref[...], bd3_ref[...]))

    x = x2 + expand(P3, x3)
    x = jax.nn.relu(gcn(x, A2_, dis2, Wu0_ref[...], bu0_ref[...]))
    x = x1 + expand(P2, x)
    x = jax.nn.relu(gcn(x, A1, dis1, Wu1_ref[...], bu1_ref[...]))
    x = x0 + expand(P1, x)
    out_ref[...] = gcn(x, ew0, dis0, Wu2_ref[...], bu2_ref[...])


@jax.jit
def kernel(X, A, W_down0, b_down0, W_down1, b_down1, W_down2, b_down2,
           W_down3, b_down3, p_pool0, p_pool1, p_pool2,
           W_up0, b_up0, W_up1, b_up1, W_up2, b_up2):
    args = (
        X, A,
        W_down0, b_down0.reshape(1, -1),
        W_down1, b_down1.reshape(1, -1),
        W_down2, b_down2.reshape(1, -1),
        W_down3, b_down3.reshape(1, -1),
        p_pool0.reshape(1, -1), p_pool1.reshape(1, -1), p_pool2.reshape(1, -1),
        W_up0, b_up0.reshape(1, -1),
        W_up1, b_up1.reshape(1, -1),
        W_up2, b_up2.reshape(1, -1),
    )
    return pl.pallas_call(
        _gunet_body,
        out_shape=jax.ShapeDtypeStruct((N, W_up2.shape[1]), F32),
    )(*args)


# speedup vs baseline: 6762.6866x; 1.0767x over previous
import jax
import jax.numpy as jnp
from jax import lax
from jax.experimental import pallas as pl

N = 1024
F32 = jnp.float32
BF16 = jnp.bfloat16


def _mm(a, b, ca, cb):
    return lax.dot_general(a, b, (((ca,), (cb,)), ((), ())),
                           preferred_element_type=F32)


def _gunet_body(X_ref, A_ref,
                Wd0_ref, bd0_ref, Wd1_ref, bd1_ref, Wd2_ref, bd2_ref,
                Wd3_ref, bd3_ref, p0_ref, p1_ref, p2_ref,
                Wu0_ref, bu0_ref, Wu1_ref, bu1_ref, Wu2_ref, bu2_ref,
                out_ref):
    X = X_ref[...]
    A = A_ref[...]

    def dis_of(Aadj):
        n = Aadj.shape[0]
        ones_col = jnp.ones((n, 1), F32)
        deg = lax.dot_general(Aadj, ones_col, (((0,), (0,)), ((), ())), preferred_element_type=F32) + 2.0
        return 1.0 / jnp.sqrt(deg)

    def gcn(x, Aadj, dis, W, b):
        h = lax.dot_general(x, W, (((1,), (0,)), ((), ())), preferred_element_type=F32)
        core = lax.dot_general(Aadj, dis * h, (((0,), (0,)), ((), ())), preferred_element_type=F32)
        return dis * core + (2.0 * dis * dis) * h + b

    def pool(x, Acur, p, binary=False):
        n = x.shape[0]
        k = n // 2
        ri = lax.broadcasted_iota(jnp.int32, (n, n), 0)
        ci = lax.broadcasted_iota(jnp.int32, (n, n), 1)
        eye_b = ri == ci
        Ap = jnp.where(eye_b, 1.0, Acur)
        if binary:
            Ap = Ap.astype(BF16)
        A2 = lax.dot_general(Ap, Ap, (((1,), (0,)), ((), ())), preferred_element_type=F32)
        nrm = jnp.sqrt(_mm(p, p, 1, 1))
        s = jnp.tanh(lax.dot_general(x, p, (((1,), (1,)), ((), ())), preferred_element_type=F32) / nrm)
        s_row = jnp.transpose(s)
        beats = ((s_row > s) | ((s_row == s) & (ci < ri))).astype(BF16)
        rank = lax.dot_general(beats, jnp.ones((n, 1), BF16),
                               (((1,), (0,)), ((), ())), preferred_element_type=F32)
        P = (jnp.transpose(rank) == ri.astype(F32)).astype(F32)[:k, :]
        xc = lax.dot_general(P, x * s, (((1,), (0,)), ((), ())), preferred_element_type=F32)
        PA2 = lax.dot_general(P, A2, (((1,), (0,)), ((), ())), preferred_element_type=F32)
        Ac = lax.dot_general(PA2, P, (((1,), (1,)), ((), ())), preferred_element_type=F32)
        eye_k = (lax.broadcasted_iota(jnp.int32, (k, k), 0)
                 == lax.broadcasted_iota(jnp.int32, (k, k), 1))
        Ac = jnp.where(eye_k, 0.0, Ac)
        return xc, Ac, P

    def expand(P, xc):
        return lax.dot_general(P, xc, (((0,), (0,)), ((), ())), preferred_element_type=F32)

    ew0 = jnp.where(A != 0, 1.0, 0.0).astype(F32)
    dis0 = dis_of(ew0)
    x0 = jax.nn.relu(gcn(X, ew0, dis0, Wd0_ref[...], bd0_ref[...]))

    xp, A1, P1 = pool(x0, ew0, p0_ref[...], binary=True)
    dis1 = dis_of(A1)
    x1 = jax.nn.relu(gcn(xp, A1, dis1, Wd1_ref[...], bd1_ref[...]))

    xp, A2_, P2 = pool(x1, A1, p1_ref[...])
    dis2 = dis_of(A2_)
    x2 = jax.nn.relu(gcn(xp, A2_, dis2, Wd2_ref[...], bd2_ref[...]))

    xp, A3, P3 = pool(x2, A2_, p2_ref[...])
    dis3 = dis_of(A3)
    x3 = jax.nn.relu(gcn(xp, A3, dis3, Wd3_ref[...], bd3_ref[...]))

    x = x2 + expand(P3, x3)
    x = jax.nn.relu(gcn(x, A2_, dis2, Wu0_ref[...], bu0_ref[...]))
    x = x1 + expand(P2, x)
    x = jax.nn.relu(gcn(x, A1, dis1, Wu1_ref[...], bu1_ref[...]))
    x = x0 + expand(P1, x)
    out_ref[...] = gcn(x, ew0, dis0, Wu2_ref[...], bu2_ref[...])


@jax.jit
def kernel(X, A, W_down0, b_down0, W_down1, b_down1, W_down2, b_down2,
           W_down3, b_down3, p_pool0, p_pool1, p_pool2,
           W_up0, b_up0, W_up1, b_up1, W_up2, b_up2):
    args = (
        X, A,
        W_down0, b_down0.reshape(1, -1),
        W_down1, b_down1.reshape(1, -1),
        W_down2, b_down2.reshape(1, -1),
        W_down3, b_down3.reshape(1, -1),
        p_pool0.reshape(1, -1), p_pool1.reshape(1, -1), p_pool2.reshape(1, -1),
        W_up0, b_up0.reshape(1, -1),
        W_up1, b_up1.reshape(1, -1),
        W_up2, b_up2.reshape(1, -1),
    )
    return pl.pallas_call(
        _gunet_body,
        out_shape=jax.ShapeDtypeStruct((N, W_up2.shape[1]), F32),
    )(*args)
